# baseline (speedup 1.0000x reference)
import functools

import jax
import jax.numpy as jnp
from jax import lax
from jax.experimental import pallas as pl
from jax.experimental.pallas import tpu as pltpu
from jax.experimental.pallas import tpu_sc as plsc

B = 16384
DIM_CONTEXT = 128
EMB_DIM = 32
N_ITEMS = 1000000

NC = 2
NS = 16
NW = NC * NS
ROWS_PER_W = B // NW
GROUP = 16
NGROUP = ROWS_PER_W // GROUP
LANES = 128

_sc_mesh = plsc.VectorSubcoreMesh(core_axis_name="c", subcore_axis_name="s")


@functools.partial(
    pl.kernel,
    mesh=_sc_mesh,
    out_type=jax.ShapeDtypeStruct((EMB_DIM, B), jnp.float32),
    scratch_types=[
        pltpu.VMEM((ROWS_PER_W,), jnp.int32),
        pltpu.VMEM((GROUP * EMB_DIM, LANES), jnp.float32),
        pltpu.VMEM((EMB_DIM * ROWS_PER_W,), jnp.float32),
        pltpu.SemaphoreType.DMA,
    ],
    compiler_params=pltpu.CompilerParams(
        disable_bounds_checks=True, needs_layout_passes=False),
)
def _sc_gather(tabT_hbm, ids_hbm, embT_hbm, idx_v, blk_v, emb_v, sem):
    wid = lax.axis_index("s") * NC + lax.axis_index("c")
    base = wid * ROWS_PER_W
    pltpu.sync_copy(ids_hbm.at[pl.ds(base, ROWS_PER_W)], idx_v)

    rows_base = lax.iota(jnp.int32, GROUP) * EMB_DIM

    def group_body(g, carry):
        k0 = g * GROUP
        cols = idx_v[pl.ds(k0, GROUP)]
        off = lax.rem(cols, jnp.int32(LANES))
        copies = []
        for j in range(GROUP):
            col_al = pl.multiple_of(
                (cols[j] // LANES) * LANES, LANES)
            copies.append(pltpu.make_async_copy(
                tabT_hbm.at[:, pl.ds(col_al, LANES)],
                blk_v.at[pl.ds(j * EMB_DIM, EMB_DIM), :],
                sem,
            ))
        for c in copies:
            c.start()
        for c in copies:
            c.wait()

        for d in range(EMB_DIM):
            v = plsc.load_gather(blk_v, [rows_base + d, off])
            emb_v[pl.ds(d * ROWS_PER_W + k0, GROUP)] = v
        return carry

    lax.fori_loop(0, NGROUP, group_body, 0)
    for d in range(EMB_DIM):
        pltpu.sync_copy(
            emb_v.at[pl.ds(d * ROWS_PER_W, ROWS_PER_W)],
            embT_hbm.at[d, pl.ds(base, ROWS_PER_W)],
        )


_PBLK = 2048
_PGRID = B // _PBLK


def _tc_fuse_body(w_ref, ctx_ref, bb_ref, emb_ref, out_ref):
    proj = lax.dot_general(
        w_ref[...], ctx_ref[...], (((1,), (1,)), ((), ())),
        preferred_element_type=jnp.float32,
    )
    bias = jnp.broadcast_to(bb_ref[...][:, 0:1], (EMB_DIM, _PBLK))
    out_ref[...] = jnp.sum((proj + bias) * emb_ref[...], axis=0)


_tc_fuse = pl.pallas_call(
    _tc_fuse_body,
    grid=(_PGRID,),
    in_specs=[
        pl.BlockSpec((EMB_DIM, DIM_CONTEXT), lambda i: (0, 0)),
        pl.BlockSpec((_PBLK, DIM_CONTEXT), lambda i: (i, 0)),
        pl.BlockSpec((EMB_DIM, 128), lambda i: (0, 0)),
        pl.BlockSpec((EMB_DIM, _PBLK), lambda i: (0, i)),
    ],
    out_specs=pl.BlockSpec((_PBLK,), lambda i: (i,)),
    out_shape=jax.ShapeDtypeStruct((B,), jnp.float32),
)


def kernel(context, item_ids, W, b, table):
    ids = item_ids.astype(jnp.int32)
    bb = jnp.broadcast_to(b.reshape(EMB_DIM, 1), (EMB_DIM, 128))
    embT = _sc_gather(table.T, ids)
    return _tc_fuse(W, context, bb, embT)


# speedup vs baseline: 1.0330x; 1.0330x over previous
import functools

import jax
import jax.numpy as jnp
from jax import lax
from jax.experimental import pallas as pl
from jax.experimental.pallas import tpu as pltpu
from jax.experimental.pallas import tpu_sc as plsc

B = 16384
DIM_CONTEXT = 128
EMB_DIM = 32
N_ITEMS = 1000000

NC = 2
NS = 16
NW = NC * NS
ROWS_PER_W = B // NW
GROUP = 16
NGROUP = ROWS_PER_W // GROUP
LANES = 128

_sc_mesh = plsc.VectorSubcoreMesh(core_axis_name="c", subcore_axis_name="s")


@functools.partial(
    pl.kernel,
    mesh=_sc_mesh,
    out_type=jax.ShapeDtypeStruct((B,), jnp.float32),
    scratch_types=[
        pltpu.VMEM((ROWS_PER_W,), jnp.int32),
        pltpu.VMEM((GROUP * EMB_DIM, LANES), jnp.float32),
        pltpu.VMEM((EMB_DIM * ROWS_PER_W,), jnp.float32),
        pltpu.VMEM((ROWS_PER_W,), jnp.float32),
        pltpu.SemaphoreType.DMA,
        pltpu.SemaphoreType.DMA,
    ],
    compiler_params=pltpu.CompilerParams(
        disable_bounds_checks=True, needs_layout_passes=False),
)
def _sc_gather_dot(tabT_hbm, ids_hbm, projT_hbm, out_hbm,
                   idx_v, blk_v, pj_v, out_v, sem, psem):
    wid = lax.axis_index("s") * NC + lax.axis_index("c")
    base = wid * ROWS_PER_W
    pltpu.sync_copy(ids_hbm.at[pl.ds(base, ROWS_PER_W)], idx_v)
    pj_copies = [
        pltpu.async_copy(
            projT_hbm.at[d, pl.ds(base, ROWS_PER_W)],
            pj_v.at[pl.ds(d * ROWS_PER_W, ROWS_PER_W)],
            psem,
        )
        for d in range(EMB_DIM)
    ]
    for c in pj_copies:
        c.wait()

    rows_base = lax.iota(jnp.int32, GROUP) * EMB_DIM

    def group_body(g, carry):
        k0 = g * GROUP
        cols = idx_v[pl.ds(k0, GROUP)]
        off = lax.rem(cols, jnp.int32(LANES))
        copies = []
        for j in range(GROUP):
            col_al = pl.multiple_of(
                (cols[j] // LANES) * LANES, LANES)
            copies.append(pltpu.make_async_copy(
                tabT_hbm.at[:, pl.ds(col_al, LANES)],
                blk_v.at[pl.ds(j * EMB_DIM, EMB_DIM), :],
                sem,
            ))
        for c in copies:
            c.start()
        for c in copies:
            c.wait()

        acc = jnp.zeros((GROUP,), jnp.float32)
        for d in range(EMB_DIM):
            v = plsc.load_gather(blk_v, [rows_base + d, off])
            acc = acc + v * pj_v[pl.ds(d * ROWS_PER_W + k0, GROUP)]
        out_v[pl.ds(k0, GROUP)] = acc
        return carry

    lax.fori_loop(0, NGROUP, group_body, 0)
    pltpu.sync_copy(out_v, out_hbm.at[pl.ds(base, ROWS_PER_W)])


_PBLK = 2048
_PGRID = B // _PBLK


def _tc_proj_body(w_ref, ctx_ref, bb_ref, out_ref):
    proj = lax.dot_general(
        w_ref[...], ctx_ref[...], (((1,), (1,)), ((), ())),
        preferred_element_type=jnp.float32,
    )
    bias = jnp.broadcast_to(bb_ref[...][:, 0:1], (EMB_DIM, _PBLK))
    out_ref[...] = proj + bias


_tc_proj = pl.pallas_call(
    _tc_proj_body,
    grid=(_PGRID,),
    in_specs=[
        pl.BlockSpec((EMB_DIM, DIM_CONTEXT), lambda i: (0, 0)),
        pl.BlockSpec((_PBLK, DIM_CONTEXT), lambda i: (i, 0)),
        pl.BlockSpec((EMB_DIM, 128), lambda i: (0, 0)),
    ],
    out_specs=pl.BlockSpec((EMB_DIM, _PBLK), lambda i: (0, i)),
    out_shape=jax.ShapeDtypeStruct((EMB_DIM, B), jnp.float32),
)


def kernel(context, item_ids, W, b, table):
    ids = item_ids.astype(jnp.int32)
    bb = jnp.broadcast_to(b.reshape(EMB_DIM, 1), (EMB_DIM, 128))
    projT = _tc_proj(W, context, bb)
    return _sc_gather_dot(table.T, ids, projT)


# speedup vs baseline: 1.0371x; 1.0040x over previous
import functools

import jax
import jax.numpy as jnp
from jax import lax
from jax.experimental import pallas as pl
from jax.experimental.pallas import tpu as pltpu
from jax.experimental.pallas import tpu_sc as plsc

B = 16384
DIM_CONTEXT = 128
EMB_DIM = 32
N_ITEMS = 1000000

NC = 2
NS = 16
NW = NC * NS
ROWS_PER_W = B // NW
GROUP = 16
NGROUP = ROWS_PER_W // GROUP
LANES = 128

_sc_mesh = plsc.VectorSubcoreMesh(core_axis_name="c", subcore_axis_name="s")


@functools.partial(
    pl.kernel,
    mesh=_sc_mesh,
    out_type=jax.ShapeDtypeStruct((B,), jnp.float32),
    scratch_types=[
        pltpu.VMEM((ROWS_PER_W,), jnp.int32),
        pltpu.VMEM((GROUP * EMB_DIM, LANES), jnp.float32),
        pltpu.VMEM((EMB_DIM * ROWS_PER_W,), jnp.float32),
        pltpu.VMEM((ROWS_PER_W,), jnp.float32),
        pltpu.SemaphoreType.DMA,
        pltpu.SemaphoreType.DMA,
    ],
    compiler_params=pltpu.CompilerParams(
        disable_bounds_checks=True, needs_layout_passes=False),
)
def _sc_gather_dot(tabT_hbm, ids_hbm, projT_hbm, out_hbm,
                   idx_v, blk_v, pj_v, out_v, sem, psem):
    wid = lax.axis_index("s") * NC + lax.axis_index("c")
    base = wid * ROWS_PER_W
    pltpu.sync_copy(ids_hbm.at[pl.ds(base, ROWS_PER_W)], idx_v)
    pj_copies = [
        pltpu.async_copy(
            projT_hbm.at[d, pl.ds(base, ROWS_PER_W)],
            pj_v.at[pl.ds(d * ROWS_PER_W, ROWS_PER_W)],
            psem,
        )
        for d in range(EMB_DIM)
    ]
    for c in pj_copies:
        c.wait()

    rows_base = lax.iota(jnp.int32, GROUP) * EMB_DIM

    def group_body(g, carry):
        k0 = g * GROUP
        cols = idx_v[pl.ds(k0, GROUP)]
        off = lax.rem(cols, jnp.int32(LANES))
        copies = []
        for j in range(GROUP):
            col_al = pl.multiple_of(
                (cols[j] // LANES) * LANES, LANES)
            for r in range(EMB_DIM // 8):
                copies.append(pltpu.make_async_copy(
                    tabT_hbm.at[pl.ds(r * 8, 8), pl.ds(col_al, LANES)],
                    blk_v.at[pl.ds(j * EMB_DIM + r * 8, 8), :],
                    sem,
                ))
        for c in copies:
            c.start()
        for c in copies:
            c.wait()

        acc = jnp.zeros((GROUP,), jnp.float32)
        for d in range(EMB_DIM):
            v = plsc.load_gather(blk_v, [rows_base + d, off])
            acc = acc + v * pj_v[pl.ds(d * ROWS_PER_W + k0, GROUP)]
        out_v[pl.ds(k0, GROUP)] = acc
        return carry

    lax.fori_loop(0, NGROUP, group_body, 0)
    pltpu.sync_copy(out_v, out_hbm.at[pl.ds(base, ROWS_PER_W)])


_PBLK = 2048
_PGRID = B // _PBLK


def _tc_proj_body(w_ref, ctx_ref, bb_ref, out_ref):
    proj = lax.dot_general(
        w_ref[...], ctx_ref[...], (((1,), (1,)), ((), ())),
        preferred_element_type=jnp.float32,
    )
    bias = jnp.broadcast_to(bb_ref[...][:, 0:1], (EMB_DIM, _PBLK))
    out_ref[...] = proj + bias


_tc_proj = pl.pallas_call(
    _tc_proj_body,
    grid=(_PGRID,),
    in_specs=[
        pl.BlockSpec((EMB_DIM, DIM_CONTEXT), lambda i: (0, 0)),
        pl.BlockSpec((_PBLK, DIM_CONTEXT), lambda i: (i, 0)),
        pl.BlockSpec((EMB_DIM, 128), lambda i: (0, 0)),
    ],
    out_specs=pl.BlockSpec((EMB_DIM, _PBLK), lambda i: (0, i)),
    out_shape=jax.ShapeDtypeStruct((EMB_DIM, B), jnp.float32),
)


def kernel(context, item_ids, W, b, table):
    ids = item_ids.astype(jnp.int32)
    bb = jnp.broadcast_to(b.reshape(EMB_DIM, 1), (EMB_DIM, 128))
    projT = _tc_proj(W, context, bb)
    return _sc_gather_dot(table.T, ids, projT)


# speedup vs baseline: 1.0500x; 1.0125x over previous
import functools

import jax
import jax.numpy as jnp
from jax import lax
from jax.experimental import pallas as pl
from jax.experimental.pallas import tpu as pltpu
from jax.experimental.pallas import tpu_sc as plsc

B = 16384
DIM_CONTEXT = 128
EMB_DIM = 32
N_ITEMS = 1000000

NC = 2
NS = 16
NW = NC * NS
ROWS_PER_W = B // NW
GROUP = 16
NGROUP = ROWS_PER_W // GROUP
LANES = 128
SLOTS = 27
EARLY = SLOTS - GROUP

_sc_mesh = plsc.VectorSubcoreMesh(core_axis_name="c", subcore_axis_name="s")


@functools.partial(
    pl.kernel,
    mesh=_sc_mesh,
    out_type=jax.ShapeDtypeStruct((B,), jnp.float32),
    scratch_types=[
        pltpu.VMEM((ROWS_PER_W,), jnp.int32),
        pltpu.VMEM((SLOTS * EMB_DIM, LANES), jnp.float32),
        pltpu.VMEM((EMB_DIM * ROWS_PER_W,), jnp.float32),
        pltpu.VMEM((ROWS_PER_W,), jnp.float32),
        pltpu.SemaphoreType.DMA,
        pltpu.SemaphoreType.DMA,
        pltpu.SemaphoreType.DMA,
    ],
    compiler_params=pltpu.CompilerParams(
        disable_bounds_checks=True, needs_layout_passes=False),
)
def _sc_gather_dot(tabT_hbm, ids_hbm, projT_hbm, out_hbm,
                   idx_v, blk_v, pj_v, out_v, sem_a, sem_b, psem):
    wid = lax.axis_index("s") * NC + lax.axis_index("c")
    base = wid * ROWS_PER_W
    pltpu.sync_copy(ids_hbm.at[pl.ds(base, ROWS_PER_W)], idx_v)
    pj_copies = [
        pltpu.async_copy(
            projT_hbm.at[d, pl.ds(base, ROWS_PER_W)],
            pj_v.at[pl.ds(d * ROWS_PER_W, ROWS_PER_W)],
            psem,
        )
        for d in range(EMB_DIM)
    ]

    grp_iota = lax.iota(jnp.int32, GROUP)

    def start_items(g, j_lo, j_hi, sem):
        k0 = g * GROUP
        cols = idx_v[pl.ds(k0, GROUP)]
        for j in range(j_lo, j_hi):
            slot = lax.rem(k0 + j, jnp.int32(SLOTS))
            col_al = pl.multiple_of((cols[j] // LANES) * LANES, LANES)
            pltpu.make_async_copy(
                tabT_hbm.at[:, pl.ds(col_al, LANES)],
                blk_v.at[pl.ds(slot * EMB_DIM, EMB_DIM), :],
                sem,
            ).start()

    def wait_group(sem):
        for _ in range(GROUP):
            pltpu.make_async_copy(
                tabT_hbm.at[:, pl.ds(0, LANES)],
                blk_v.at[pl.ds(0, EMB_DIM), :],
                sem,
            ).wait()

    def compute_group(g):
        k0 = g * GROUP
        cols = idx_v[pl.ds(k0, GROUP)]
        off = lax.rem(cols, jnp.int32(LANES))
        rows = lax.rem(k0 + grp_iota, jnp.int32(SLOTS)) * EMB_DIM
        acc = jnp.zeros((GROUP,), jnp.float32)
        for d in range(EMB_DIM):
            v = plsc.load_gather(blk_v, [rows + d, off])
            acc = acc + v * pj_v[pl.ds(d * ROWS_PER_W + k0, GROUP)]
        out_v[pl.ds(k0, GROUP)] = acc

    start_items(0, 0, GROUP, sem_a)
    for c in pj_copies:
        c.wait()

    def pipe_body(i, carry):
        g0 = 2 * i
        wait_group(sem_a)
        start_items(g0 + 1, 0, EARLY, sem_b)
        compute_group(g0)
        start_items(g0 + 1, EARLY, GROUP, sem_b)
        wait_group(sem_b)
        start_items(g0 + 2, 0, EARLY, sem_a)
        compute_group(g0 + 1)
        start_items(g0 + 2, EARLY, GROUP, sem_a)
        return carry

    lax.fori_loop(0, NGROUP // 2 - 1, pipe_body, 0)

    wait_group(sem_a)
    start_items(NGROUP - 1, 0, EARLY, sem_b)
    compute_group(NGROUP - 2)
    start_items(NGROUP - 1, EARLY, GROUP, sem_b)
    wait_group(sem_b)
    compute_group(NGROUP - 1)

    pltpu.sync_copy(out_v, out_hbm.at[pl.ds(base, ROWS_PER_W)])


_PBLK = 2048
_PGRID = B // _PBLK


def _tc_proj_body(w_ref, ctx_ref, bb_ref, out_ref):
    proj = lax.dot_general(
        w_ref[...], ctx_ref[...], (((1,), (1,)), ((), ())),
        preferred_element_type=jnp.float32,
    )
    bias = jnp.broadcast_to(bb_ref[...][:, 0:1], (EMB_DIM, _PBLK))
    out_ref[...] = proj + bias


_tc_proj = pl.pallas_call(
    _tc_proj_body,
    grid=(_PGRID,),
    in_specs=[
        pl.BlockSpec((EMB_DIM, DIM_CONTEXT), lambda i: (0, 0)),
        pl.BlockSpec((_PBLK, DIM_CONTEXT), lambda i: (i, 0)),
        pl.BlockSpec((EMB_DIM, 128), lambda i: (0, 0)),
    ],
    out_specs=pl.BlockSpec((EMB_DIM, _PBLK), lambda i: (0, i)),
    out_shape=jax.ShapeDtypeStruct((EMB_DIM, B), jnp.float32),
)


def kernel(context, item_ids, W, b, table):
    ids = item_ids.astype(jnp.int32)
    bb = jnp.broadcast_to(b.reshape(EMB_DIM, 1), (EMB_DIM, 128))
    projT = _tc_proj(W, context, bb)
    return _sc_gather_dot(table.T, ids, projT)


# speedup vs baseline: 1.1057x; 1.0531x over previous
import functools

import jax
import jax.numpy as jnp
from jax import lax
from jax.experimental import pallas as pl
from jax.experimental.pallas import tpu as pltpu
from jax.experimental.pallas import tpu_sc as plsc

B = 16384
DIM_CONTEXT = 128
EMB_DIM = 32
N_ITEMS = 1000000

NC = 2
NS = 16
NW = NC * NS
ROWS_PER_W = B // NW
GROUP = 16
NGROUP = ROWS_PER_W // GROUP
LANES = 128
SLOTS = 27
EARLY = SLOTS - GROUP

_sc_mesh = plsc.VectorSubcoreMesh(core_axis_name="c", subcore_axis_name="s")


@functools.partial(
    pl.kernel,
    mesh=_sc_mesh,
    out_type=jax.ShapeDtypeStruct((B,), jnp.float32),
    scratch_types=[
        pltpu.VMEM((ROWS_PER_W,), jnp.int32),
        pltpu.VMEM((SLOTS * EMB_DIM, LANES), jnp.float32),
        pltpu.VMEM((EMB_DIM * ROWS_PER_W,), jnp.float32),
        pltpu.VMEM((ROWS_PER_W,), jnp.float32),
        pltpu.SemaphoreType.DMA,
        pltpu.SemaphoreType.DMA,
        pltpu.SemaphoreType.DMA,
    ],
    compiler_params=pltpu.CompilerParams(
        disable_bounds_checks=True, needs_layout_passes=False),
)
def _sc_gather_dot(tabT_hbm, ids_hbm, projT_hbm, out_hbm,
                   idx_v, blk_v, pj_v, out_v, sem_a, sem_b, psem):
    wid = lax.axis_index("s") * NC + lax.axis_index("c")
    base = wid * ROWS_PER_W
    pltpu.sync_copy(ids_hbm.at[pl.ds(base, ROWS_PER_W)], idx_v)
    pj_copies = [
        pltpu.async_copy(
            projT_hbm.at[d, pl.ds(base, ROWS_PER_W)],
            pj_v.at[pl.ds(d * ROWS_PER_W, ROWS_PER_W)],
            psem,
        )
        for d in range(EMB_DIM)
    ]

    grp_iota = lax.iota(jnp.int32, GROUP)

    def start_items(g, j_lo, j_hi, sem):
        k0 = g * GROUP
        cols = idx_v[pl.ds(k0, GROUP)]
        for j in range(j_lo, j_hi):
            slot = lax.rem(k0 + j, jnp.int32(SLOTS))
            col_al = pl.multiple_of((cols[j] // LANES) * LANES, LANES)
            pltpu.make_async_copy(
                tabT_hbm.at[:, pl.ds(col_al, LANES)],
                blk_v.at[pl.ds(slot * EMB_DIM, EMB_DIM), :],
                sem,
            ).start()

    def wait_group(sem):
        for _ in range(GROUP):
            pltpu.make_async_copy(
                tabT_hbm.at[:, pl.ds(0, LANES)],
                blk_v.at[pl.ds(0, EMB_DIM), :],
                sem,
            ).wait()

    def compute_group(g):
        k0 = g * GROUP
        cols = idx_v[pl.ds(k0, GROUP)]
        off = lax.rem(cols, jnp.int32(LANES))
        rows = lax.rem(k0 + grp_iota, jnp.int32(SLOTS)) * EMB_DIM
        acc = jnp.zeros((GROUP,), jnp.float32)
        for d in range(EMB_DIM):
            v = plsc.load_gather(blk_v, [rows + d, off])
            acc = acc + v * pj_v[pl.ds(d * ROWS_PER_W + k0, GROUP)]
        out_v[pl.ds(k0, GROUP)] = acc

    start_items(0, 0, GROUP, sem_a)
    for c in pj_copies:
        c.wait()

    def pipe_body(i, carry):
        g0 = 2 * i
        start_items(g0 + 1, 0, EARLY, sem_b)
        wait_group(sem_a)
        compute_group(g0)
        start_items(g0 + 1, EARLY, GROUP, sem_b)
        start_items(g0 + 2, 0, EARLY, sem_a)
        wait_group(sem_b)
        compute_group(g0 + 1)
        start_items(g0 + 2, EARLY, GROUP, sem_a)
        return carry

    lax.fori_loop(0, NGROUP // 2 - 1, pipe_body, 0)

    start_items(NGROUP - 1, 0, EARLY, sem_b)
    wait_group(sem_a)
    compute_group(NGROUP - 2)
    start_items(NGROUP - 1, EARLY, GROUP, sem_b)
    wait_group(sem_b)
    compute_group(NGROUP - 1)

    pltpu.sync_copy(out_v, out_hbm.at[pl.ds(base, ROWS_PER_W)])


_PBLK = 2048
_PGRID = B // _PBLK


def _tc_proj_body(w_ref, ctx_ref, bb_ref, out_ref):
    proj = lax.dot_general(
        w_ref[...], ctx_ref[...], (((1,), (1,)), ((), ())),
        preferred_element_type=jnp.float32,
    )
    bias = jnp.broadcast_to(bb_ref[...][:, 0:1], (EMB_DIM, _PBLK))
    out_ref[...] = proj + bias


_tc_proj = pl.pallas_call(
    _tc_proj_body,
    grid=(_PGRID,),
    in_specs=[
        pl.BlockSpec((EMB_DIM, DIM_CONTEXT), lambda i: (0, 0)),
        pl.BlockSpec((_PBLK, DIM_CONTEXT), lambda i: (i, 0)),
        pl.BlockSpec((EMB_DIM, 128), lambda i: (0, 0)),
    ],
    out_specs=pl.BlockSpec((EMB_DIM, _PBLK), lambda i: (0, i)),
    out_shape=jax.ShapeDtypeStruct((EMB_DIM, B), jnp.float32),
)


def kernel(context, item_ids, W, b, table):
    ids = item_ids.astype(jnp.int32)
    bb = jnp.broadcast_to(b.reshape(EMB_DIM, 1), (EMB_DIM, 128))
    projT = _tc_proj(W, context, bb)
    return _sc_gather_dot(table.T, ids, projT)
